# Initial kernel scaffold; baseline (speedup 1.0000x reference)
import jax
import jax.numpy as jnp
from jax.experimental import pallas as pl


def kernel(z, embedding):
    raise NotImplementedError("write your pallas kernel here")


# speedup vs baseline: 67.9579x; 67.9579x over previous
import functools

import jax
import jax.numpy as jnp
from jax import lax
from jax.experimental import pallas as pl
from jax.experimental.pallas import tpu as pltpu
from jax.experimental.pallas import tpu_sc as plsc

_N_E = 8192
_D = 64
_B = 8
_N = 576
_ROWS = _B * _N

_T = 1024
_CB = _N_E // _T

_RB = 512
_NB = _ROWS // _RB
_T2 = 2048
_CB2 = _N_E // _T2


def _k0_normalize(emb_ref, out_ref):
    e = emb_ref[...]
    nrm = jnp.sqrt(jnp.sum(e * e, axis=1, keepdims=True))
    out_ref[:, :_D] = e / jnp.maximum(nrm, 1e-12)
    out_ref[:, _D:] = jnp.zeros((_N_E, 128 - _D), jnp.float32)


def _k1_argmax(z_ref, emb_ref, out_ref, bv_ref, bi_ref):
    cb = pl.program_id(1)
    zt = z_ref[0]
    znrm = jnp.sqrt(jnp.sum(zt * zt, axis=0, keepdims=True))
    zn = zt / jnp.maximum(znrm, 1e-12)
    zz = jnp.sum(zn * zn, axis=0, keepdims=True)

    et = emb_ref[...]
    enrm = jnp.sqrt(jnp.sum(et * et, axis=1, keepdims=True))
    en = et / jnp.maximum(enrm, 1e-12)
    ee = jnp.sum(en * en, axis=1, keepdims=True)

    dots = jnp.dot(en, zn, preferred_element_type=jnp.float32)
    d = (-zz - ee) + 2.0 * dots

    m = jnp.max(d, axis=0, keepdims=True)
    rid = lax.broadcasted_iota(jnp.int32, (_T, _N), 0) + cb * _T
    cand = jnp.max(jnp.where(d == m, rid, -1), axis=0, keepdims=True)

    @pl.when(cb == 0)
    def _():
        bv_ref[...] = m
        bi_ref[...] = cand

    @pl.when(cb > 0)
    def _():
        upd = m >= bv_ref[...]
        bv_ref[...] = jnp.where(upd, m, bv_ref[...])
        bi_ref[...] = jnp.where(upd, cand, bi_ref[...])

    @pl.when(cb == _CB - 1)
    def _():
        out_ref[...] = bi_ref[...][None]


def _k2_onehot(idx_ref, oh_ref, cnt_ref):
    cb = pl.program_id(0)
    rb = pl.program_id(1)
    idxb = idx_ref[0]
    col = lax.broadcasted_iota(jnp.int32, (_RB, _T2), 1) + cb * _T2
    oh = (idxb == col).astype(jnp.float32)
    oh_ref[...] = oh
    colsum = jnp.sum(oh, axis=0, keepdims=True)

    @pl.when(rb == 0)
    def _():
        cnt_ref[...] = colsum

    @pl.when(rb > 0)
    def _():
        cnt_ref[...] = cnt_ref[...] + colsum


def _k4_perplexity(cnt_ref, out_ref):
    p = cnt_ref[...] / float(_ROWS)
    t = p * jnp.log(p + 1e-10)
    s = jnp.sum(t, axis=(0, 1), keepdims=True)
    out_ref[...] = jnp.exp(-s)


def _sc_gather(emb_pad, idx_flat):
    info = plsc.get_sparse_core_info()
    nc, ns = info.num_cores, info.num_subcores
    nw = nc * ns
    bpw = _ROWS // nw
    ch = 72
    nch = bpw // ch
    idx2 = idx_flat.reshape(_ROWS // ch, ch)
    mesh = plsc.VectorSubcoreMesh(core_axis_name="c", subcore_axis_name="s")

    @functools.partial(
        pl.kernel,
        mesh=mesh,
        out_type=jax.ShapeDtypeStruct((_ROWS, 128), jnp.float32),
        scratch_types=[
            pltpu.VMEM((nch, ch), jnp.int32),
            pltpu.VMEM((bpw, 128), jnp.float32),
            pltpu.SemaphoreType.DMA,
        ],
    )
    def gather_k(emb_hbm, idx_hbm, out_hbm, idx_v, rows_v, sem):
        wid = lax.axis_index("s") * nc + lax.axis_index("c")
        base = wid * bpw
        pltpu.sync_copy(idx_hbm.at[pl.ds(wid * nch, nch)], idx_v)
        cps = [
            pltpu.async_copy(
                emb_hbm.at[idx_v.at[j]], rows_v.at[pl.ds(j * ch, ch)], sem
            )
            for j in range(nch)
        ]
        for cp in cps:
            cp.wait()
        pltpu.sync_copy(rows_v, out_hbm.at[pl.ds(base, bpw)])

    return gather_k(emb_pad, idx2)


def kernel(z, embedding):
    emb_pad = pl.pallas_call(
        _k0_normalize,
        out_shape=jax.ShapeDtypeStruct((_N_E, 128), jnp.float32),
    )(embedding)

    idx3 = pl.pallas_call(
        _k1_argmax,
        grid=(_B, _CB),
        in_specs=[
            pl.BlockSpec((1, _D, _N), lambda b, cb: (b, 0, 0)),
            pl.BlockSpec((_T, _D), lambda b, cb: (cb, 0)),
        ],
        out_specs=pl.BlockSpec((1, 1, _N), lambda b, cb: (b, 0, 0)),
        out_shape=jax.ShapeDtypeStruct((_B, 1, _N), jnp.int32),
        scratch_shapes=[
            pltpu.VMEM((1, _N), jnp.float32),
            pltpu.VMEM((1, _N), jnp.int32),
        ],
    )(z, embedding)
    indices = idx3.reshape(_ROWS)

    onehot, counts = pl.pallas_call(
        _k2_onehot,
        grid=(_CB2, _NB),
        in_specs=[
            pl.BlockSpec((1, _RB, 1), lambda cb, rb: (rb, 0, 0)),
        ],
        out_specs=[
            pl.BlockSpec((_RB, _T2), lambda cb, rb: (rb, cb)),
            pl.BlockSpec((1, _T2), lambda cb, rb: (0, cb)),
        ],
        out_shape=[
            jax.ShapeDtypeStruct((_ROWS, _N_E), jnp.float32),
            jax.ShapeDtypeStruct((1, _N_E), jnp.float32),
        ],
    )(indices.reshape(_NB, _RB, 1))

    perp = pl.pallas_call(
        _k4_perplexity,
        out_shape=jax.ShapeDtypeStruct((1, 1), jnp.float32),
    )(counts).reshape(())

    zq = _sc_gather(emb_pad, indices)[:, :_D]
    quant = zq.reshape(_B, _N, _D).transpose(0, 2, 1)

    zero = jnp.float32(0.0)
    return (quant, zero, zero, zero, zero, perp, onehot, indices)
